# baseline (speedup 1.0000x reference)
import functools

import jax
import jax.numpy as jnp
from jax import lax
from jax.experimental import pallas as pl
from jax.experimental.pallas import tpu as pltpu
from jax.experimental.pallas import tpu_sc as plsc

B = 16384
VOCAB = 100000
EMB = 106
N_FIELDS = 9
N_SPARSE = 3
N_BEHAVIOR = 5
DENSE = 13

_NC = 2
_NS = 16
_NW = _NC * _NS
_CHUNK = 128
_CPW = B // (_NW * _CHUNK)
_RPW = B // _NW


EMB_P = 128


def _sc_gather_body(idx_hbm, tab, out_hbm, idx_v, rows_v, sem):
    wid = lax.axis_index("s") * _NC + lax.axis_index("c")
    c0 = wid * _CPW
    pltpu.sync_copy(idx_hbm.at[wid, :], idx_v)

    def chunk_body(j, _):
        pltpu.async_copy(
            tab.at[idx_v.at[pl.ds(j * _CHUNK, _CHUNK)]], rows_v, sem).wait()
        pltpu.sync_copy(
            rows_v, out_hbm.at[pl.ds((c0 + j) * _CHUNK, _CHUNK), :])
        return 0

    lax.fori_loop(0, _CPW, chunk_body, 0)


@functools.cache
def _sc_gather():
    return pl.kernel(
        _sc_gather_body,
        out_type=jax.ShapeDtypeStruct((B, EMB_P), jnp.float32),
        mesh=plsc.VectorSubcoreMesh(core_axis_name="c", subcore_axis_name="s"),
        scratch_types=[
            pltpu.VMEM((_RPW,), jnp.int32),
            pltpu.VMEM((_CHUNK, EMB_P), jnp.float32),
            pltpu.SemaphoreType.DMA,
        ],
    )


_BB = 1024
_NB = B // _BB
_BB5 = N_BEHAVIOR * _BB


def _prelu(x, a):
    return jnp.maximum(x, 0.0) + a * jnp.minimum(x, 0.0)


def _dotT(x, w):
    return lax.dot_general(x, w, (((1,), (1,)), ((), ())),
                           precision=lax.Precision.HIGHEST,
                           preferred_element_type=jnp.float32)


def _tc_body(g0, g1, g2, g3, g4, g5, g6, g7, g8, dense_ref,
             aw1a, aw1b, aw1c, aw1d, ab1, aw2,
             w1s0, w1s1, w1s2, w1sd, w1sp, w1sc, b1, w2, b2, w3, scal,
             out_ref):
    a_att = scal[0]
    ab2 = scal[1]
    p1 = scal[2]
    p2 = scal[3]
    b3 = scal[4]

    cand = g8[...]
    amat = aw1a[...] + aw1c[...]
    bmat = aw1b[...] - aw1c[...]
    dmat = aw1d[...]
    c0 = _dotT(cand, bmat) + ab1[...]

    beh5 = jnp.stack([g3[...], g4[...], g5[...], g6[...], g7[...]], axis=0)
    behf = beh5.reshape(_BB5, EMB_P)
    bcf = (beh5 * cand[None]).reshape(_BB5, EMB_P)
    c05 = jnp.broadcast_to(c0[None], (N_BEHAVIOR, _BB, 32)).reshape(_BB5, 32)
    h = _prelu(_dotT(behf, amat) + _dotT(bcf, dmat) + c05, a_att)
    score = jnp.sum(h * aw2[...], axis=1, keepdims=True) + ab2
    att5 = jax.nn.sigmoid(score).reshape(N_BEHAVIOR, _BB, 1)
    pool = jnp.sum(att5 * beh5, axis=0)

    acc = (_dotT(g0[...], w1s0[...]) + _dotT(g1[...], w1s1[...])
           + _dotT(g2[...], w1s2[...]) + _dotT(dense_ref[...], w1sd[...])
           + _dotT(pool, w1sp[...]) + _dotT(cand, w1sc[...]) + b1[...])
    o1 = _prelu(acc, p1)
    o2 = _prelu(_dotT(o1, w2[...]) + b2[...], p2)
    o3 = jnp.sum(o2 * w3[...], axis=1) + b3
    out_ref[0, 0, :] = jax.nn.sigmoid(o3)


def _full(shape):
    n = len(shape)
    return pl.BlockSpec(shape, lambda i, _n=n: (0,) * _n)


def _make_tc_call(interpret=False):
    gspec = pl.BlockSpec((_BB, EMB_P), lambda i: (i, 0))
    return pl.pallas_call(
        _tc_body,
        grid=(_NB,),
        in_specs=[gspec] * N_FIELDS + [
            pl.BlockSpec((_BB, DENSE), lambda i: (i, 0)),
            _full((32, EMB_P)), _full((32, EMB_P)), _full((32, EMB_P)),
            _full((32, EMB_P)), _full((1, 32)), _full((1, 32)),
            _full((128, EMB_P)), _full((128, EMB_P)), _full((128, EMB_P)),
            _full((128, DENSE)), _full((128, EMB_P)), _full((128, EMB_P)),
            _full((1, 128)), _full((64, 128)), _full((1, 64)),
            _full((1, 64)),
            pl.BlockSpec(memory_space=pltpu.SMEM),
        ],
        out_specs=pl.BlockSpec((1, 1, _BB), lambda i: (i, 0, 0)),
        out_shape=jax.ShapeDtypeStruct((_NB, 1, _BB), jnp.float32),
        interpret=interpret,
    )


def _padw(w):
    return jnp.pad(w, ((0, 0), (0, EMB_P - w.shape[1])))


def _dense_stage(gathered, dense_feature, att_w1, att_b1, att_a, att_w2,
                 att_b2, w1, b1, p1, w2, b2, p2, w3, b3, interpret=False):
    scal = jnp.concatenate([
        att_a.reshape(-1), att_b2.reshape(-1), p1.reshape(-1),
        p2.reshape(-1), b3.reshape(-1),
        jnp.zeros((3,), jnp.float32)]).astype(jnp.float32)
    out = _make_tc_call(interpret)(
        *gathered, dense_feature,
        _padw(att_w1[:, :EMB]), _padw(att_w1[:, EMB:2 * EMB]),
        _padw(att_w1[:, 2 * EMB:3 * EMB]), _padw(att_w1[:, 3 * EMB:]),
        att_b1.reshape(1, 32), att_w2.reshape(1, 32),
        _padw(w1[:, :EMB]), _padw(w1[:, EMB:2 * EMB]),
        _padw(w1[:, 2 * EMB:3 * EMB]),
        w1[:, 3 * EMB:3 * EMB + DENSE],
        _padw(w1[:, 3 * EMB + DENSE:4 * EMB + DENSE]),
        _padw(w1[:, 4 * EMB + DENSE:]),
        b1.reshape(1, 128), w2, b2.reshape(1, 64), w3, scal)
    return out.reshape(-1)


def kernel(sparse_feature, dense_feature, emb_tables, att_w1, att_b1, att_a,
           att_w2, att_b2, w1, b1, p1, w2, b2, p2, w3, b3):
    idx = sparse_feature.astype(jnp.int32).T.reshape(N_FIELDS, _NW, _RPW)
    gathered = []
    for f in range(N_FIELDS):
        tab = jnp.pad(emb_tables[f], ((0, 0), (0, EMB_P - EMB)))
        gathered.append(_sc_gather()(idx[f], tab))
    return _dense_stage(gathered, dense_feature, att_w1, att_b1, att_a,
                        att_w2, att_b2, w1, b1, p1, w2, b2, p2, w3, b3)


# speedup vs baseline: 2.0938x; 2.0938x over previous
import functools

import jax
import jax.numpy as jnp
from jax import lax
from jax.experimental import pallas as pl
from jax.experimental.pallas import tpu as pltpu
from jax.experimental.pallas import tpu_sc as plsc

B = 16384
VOCAB = 100000
EMB = 106
N_FIELDS = 9
N_SPARSE = 3
N_BEHAVIOR = 5
DENSE = 13

_NC = 2
_NS = 16
_NW = _NC * _NS
_CHUNK = 128
_CPW = B // (_NW * _CHUNK)
_RPW = B // _NW


EMB_P = 128
_PR = 10000
_PNB = VOCAB // _PR


def _pad_body(i_ref, o_ref):
    o_ref[:, :EMB] = i_ref[...]


@functools.cache
def _pad_call():
    return pl.pallas_call(
        _pad_body,
        grid=(_PNB,),
        in_specs=[pl.BlockSpec((_PR, EMB), lambda i: (i, 0))],
        out_specs=pl.BlockSpec((_PR, EMB_P), lambda i: (i, 0)),
        out_shape=jax.ShapeDtypeStruct((VOCAB, EMB_P), jnp.float32),
    )


def _sc_gather_body(idx_hbm, tab, out_hbm, idx_v, rows_v, sem):
    wid = lax.axis_index("s") * _NC + lax.axis_index("c")
    c0 = wid * _CPW
    pltpu.sync_copy(idx_hbm.at[wid, :], idx_v)

    def chunk_body(j, _):
        pltpu.async_copy(
            tab.at[idx_v.at[pl.ds(j * _CHUNK, _CHUNK)]], rows_v, sem).wait()
        pltpu.sync_copy(
            rows_v, out_hbm.at[pl.ds((c0 + j) * _CHUNK, _CHUNK), :])
        return 0

    lax.fori_loop(0, _CPW, chunk_body, 0)


@functools.cache
def _sc_gather():
    return pl.kernel(
        _sc_gather_body,
        out_type=jax.ShapeDtypeStruct((B, EMB_P), jnp.float32),
        mesh=plsc.VectorSubcoreMesh(core_axis_name="c", subcore_axis_name="s"),
        scratch_types=[
            pltpu.VMEM((_RPW,), jnp.int32),
            pltpu.VMEM((_CHUNK, EMB_P), jnp.float32),
            pltpu.SemaphoreType.DMA,
        ],
    )


_BB = 1024
_NB = B // _BB
_BB5 = N_BEHAVIOR * _BB


def _prelu(x, a):
    return jnp.maximum(x, 0.0) + a * jnp.minimum(x, 0.0)


def _dotT(x, w):
    return lax.dot_general(x, w, (((1,), (1,)), ((), ())),
                           precision=lax.Precision.HIGHEST,
                           preferred_element_type=jnp.float32)


def _tc_body(g0, g1, g2, g3, g4, g5, g6, g7, g8, dense_ref,
             aw1a, aw1b, aw1c, aw1d, ab1, aw2,
             w1s0, w1s1, w1s2, w1sd, w1sp, w1sc, b1, w2, b2, w3, scal,
             out_ref):
    a_att = scal[0]
    ab2 = scal[1]
    p1 = scal[2]
    p2 = scal[3]
    b3 = scal[4]

    lane = lax.broadcasted_iota(jnp.int32, (1, EMB_P), 1)
    lmask = lane < EMB

    def _mg(g):
        return jnp.where(lmask, g[...], 0.0)

    g0m, g1m, g2m = _mg(g0), _mg(g1), _mg(g2)
    cand = _mg(g8)
    amat = aw1a[...] + aw1c[...]
    bmat = aw1b[...] - aw1c[...]
    dmat = aw1d[...]
    c0 = _dotT(cand, bmat) + ab1[...]

    beh5 = jnp.stack([_mg(g3), _mg(g4), _mg(g5), _mg(g6), _mg(g7)], axis=0)
    behf = beh5.reshape(_BB5, EMB_P)
    bcf = (beh5 * cand[None]).reshape(_BB5, EMB_P)
    c05 = jnp.broadcast_to(c0[None], (N_BEHAVIOR, _BB, 32)).reshape(_BB5, 32)
    h = _prelu(_dotT(behf, amat) + _dotT(bcf, dmat) + c05, a_att)
    score = jnp.sum(h * aw2[...], axis=1, keepdims=True) + ab2
    att5 = jax.nn.sigmoid(score).reshape(N_BEHAVIOR, _BB, 1)
    pool = jnp.sum(att5 * beh5, axis=0)

    acc = (_dotT(g0m, w1s0[...]) + _dotT(g1m, w1s1[...])
           + _dotT(g2m, w1s2[...]) + _dotT(dense_ref[...], w1sd[...])
           + _dotT(pool, w1sp[...]) + _dotT(cand, w1sc[...]) + b1[...])
    o1 = _prelu(acc, p1)
    o2 = _prelu(_dotT(o1, w2[...]) + b2[...], p2)
    o3 = jnp.sum(o2 * w3[...], axis=1) + b3
    out_ref[0, 0, :] = jax.nn.sigmoid(o3)


def _full(shape):
    n = len(shape)
    return pl.BlockSpec(shape, lambda i, _n=n: (0,) * _n)


def _make_tc_call(interpret=False):
    gspec = pl.BlockSpec((_BB, EMB_P), lambda i: (i, 0))
    return pl.pallas_call(
        _tc_body,
        grid=(_NB,),
        in_specs=[gspec] * N_FIELDS + [
            pl.BlockSpec((_BB, DENSE), lambda i: (i, 0)),
            _full((32, EMB_P)), _full((32, EMB_P)), _full((32, EMB_P)),
            _full((32, EMB_P)), _full((1, 32)), _full((1, 32)),
            _full((128, EMB_P)), _full((128, EMB_P)), _full((128, EMB_P)),
            _full((128, DENSE)), _full((128, EMB_P)), _full((128, EMB_P)),
            _full((1, 128)), _full((64, 128)), _full((1, 64)),
            _full((1, 64)),
            pl.BlockSpec(memory_space=pltpu.SMEM),
        ],
        out_specs=pl.BlockSpec((1, 1, _BB), lambda i: (i, 0, 0)),
        out_shape=jax.ShapeDtypeStruct((_NB, 1, _BB), jnp.float32),
        interpret=interpret,
    )


def _padw(w):
    return jnp.pad(w, ((0, 0), (0, EMB_P - w.shape[1])))


def _dense_stage(gathered, dense_feature, att_w1, att_b1, att_a, att_w2,
                 att_b2, w1, b1, p1, w2, b2, p2, w3, b3, interpret=False):
    scal = jnp.concatenate([
        att_a.reshape(-1), att_b2.reshape(-1), p1.reshape(-1),
        p2.reshape(-1), b3.reshape(-1),
        jnp.zeros((3,), jnp.float32)]).astype(jnp.float32)
    out = _make_tc_call(interpret)(
        *gathered, dense_feature,
        _padw(att_w1[:, :EMB]), _padw(att_w1[:, EMB:2 * EMB]),
        _padw(att_w1[:, 2 * EMB:3 * EMB]), _padw(att_w1[:, 3 * EMB:]),
        att_b1.reshape(1, 32), att_w2.reshape(1, 32),
        _padw(w1[:, :EMB]), _padw(w1[:, EMB:2 * EMB]),
        _padw(w1[:, 2 * EMB:3 * EMB]),
        w1[:, 3 * EMB:3 * EMB + DENSE],
        _padw(w1[:, 3 * EMB + DENSE:4 * EMB + DENSE]),
        _padw(w1[:, 4 * EMB + DENSE:]),
        b1.reshape(1, 128), w2, b2.reshape(1, 64), w3, scal)
    return out.reshape(-1)


def kernel(sparse_feature, dense_feature, emb_tables, att_w1, att_b1, att_a,
           att_w2, att_b2, w1, b1, p1, w2, b2, p2, w3, b3):
    idx = sparse_feature.astype(jnp.int32).T.reshape(N_FIELDS, _NW, _RPW)
    gathered = []
    for f in range(N_FIELDS):
        tab = _pad_call()(emb_tables[f])
        gathered.append(_sc_gather()(idx[f], tab))
    return _dense_stage(gathered, dense_feature, att_w1, att_b1, att_a,
                        att_w2, att_b2, w1, b1, p1, w2, b2, p2, w3, b3)


# speedup vs baseline: 2.4427x; 1.1666x over previous
import functools

import jax
import jax.numpy as jnp
from jax import lax
from jax.experimental import pallas as pl
from jax.experimental.pallas import tpu as pltpu
from jax.experimental.pallas import tpu_sc as plsc

B = 16384
VOCAB = 100000
EMB = 106
N_FIELDS = 9
N_SPARSE = 3
N_BEHAVIOR = 5
DENSE = 13

_NC = 2
_NS = 16
_NW = _NC * _NS
_CHUNK = 128
_CPW = B // (_NW * _CHUNK)
_RPW = B // _NW


EMB_P = 128
_PR = 10000
_PNB = VOCAB // _PR


def _pad_body(i_ref, o_ref):
    o_ref[:, :EMB] = i_ref[...]


@functools.cache
def _pad_call():
    return pl.pallas_call(
        _pad_body,
        grid=(_PNB,),
        in_specs=[pl.BlockSpec((_PR, EMB), lambda i: (i, 0))],
        out_specs=pl.BlockSpec((_PR, EMB_P), lambda i: (i, 0)),
        out_shape=jax.ShapeDtypeStruct((VOCAB, EMB_P), jnp.float32),
    )


def _sc_gather_body(idx_hbm, tab, out_hbm, idx_v, rows_v, sem):
    wid = lax.axis_index("s") * _NC + lax.axis_index("c")
    c0 = wid * _CPW
    pltpu.sync_copy(idx_hbm.at[wid, :], idx_v)

    def chunk_body(j, _):
        pltpu.async_copy(
            tab.at[idx_v.at[pl.ds(j * _CHUNK, _CHUNK)]], rows_v, sem).wait()
        pltpu.sync_copy(
            rows_v, out_hbm.at[pl.ds((c0 + j) * _CHUNK, _CHUNK), :])
        return 0

    lax.fori_loop(0, _CPW, chunk_body, 0)


@functools.cache
def _sc_gather():
    return pl.kernel(
        _sc_gather_body,
        out_type=jax.ShapeDtypeStruct((B, EMB_P), jnp.float32),
        mesh=plsc.VectorSubcoreMesh(core_axis_name="c", subcore_axis_name="s"),
        scratch_types=[
            pltpu.VMEM((_RPW,), jnp.int32),
            pltpu.VMEM((_CHUNK, EMB_P), jnp.float32),
            pltpu.SemaphoreType.DMA,
        ],
    )


_BB = 1024
_NB = B // _BB
_BB5 = N_BEHAVIOR * _BB


def _prelu(x, a):
    return jnp.maximum(x, 0.0) + a * jnp.minimum(x, 0.0)


def _dotT(x, w):
    return lax.dot_general(x, w, (((1,), (1,)), ((), ())),
                           preferred_element_type=jnp.float32)


def _tc_body(g0, g1, g2, g3, g4, g5, g6, g7, g8, dense_ref,
             aw1a, aw1b, aw1c, aw1d, ab1, aw2,
             w1s0, w1s1, w1s2, w1sd, w1sp, w1sc, b1, w2, b2, w3, scal,
             out_ref):
    a_att = scal[0]
    ab2 = scal[1]
    p1 = scal[2]
    p2 = scal[3]
    b3 = scal[4]

    lane = lax.broadcasted_iota(jnp.int32, (1, EMB_P), 1)
    lmask = lane < EMB

    def _mg(g):
        return jnp.where(lmask, g[...], 0.0)

    g0m, g1m, g2m = _mg(g0), _mg(g1), _mg(g2)
    cand = _mg(g8)
    amat = aw1a[...] + aw1c[...]
    bmat = aw1b[...] - aw1c[...]
    dmat = aw1d[...]
    c0 = _dotT(cand, bmat) + ab1[...]

    beh5 = jnp.stack([_mg(g3), _mg(g4), _mg(g5), _mg(g6), _mg(g7)], axis=0)
    behf = beh5.reshape(_BB5, EMB_P)
    bcf = (beh5 * cand[None]).reshape(_BB5, EMB_P)
    c05 = jnp.broadcast_to(c0[None], (N_BEHAVIOR, _BB, 32)).reshape(_BB5, 32)
    h = _prelu(_dotT(behf, amat) + _dotT(bcf, dmat) + c05, a_att)
    score = jnp.sum(h * aw2[...], axis=1, keepdims=True) + ab2
    att5 = jax.nn.sigmoid(score).reshape(N_BEHAVIOR, _BB, 1)
    pool = jnp.sum(att5 * beh5, axis=0)

    acc = (_dotT(g0m, w1s0[...]) + _dotT(g1m, w1s1[...])
           + _dotT(g2m, w1s2[...]) + _dotT(dense_ref[...], w1sd[...])
           + _dotT(pool, w1sp[...]) + _dotT(cand, w1sc[...]) + b1[...])
    o1 = _prelu(acc, p1)
    o2 = _prelu(_dotT(o1, w2[...]) + b2[...], p2)
    o3 = jnp.sum(o2 * w3[...], axis=1) + b3
    out_ref[0, 0, :] = jax.nn.sigmoid(o3)


def _full(shape):
    n = len(shape)
    return pl.BlockSpec(shape, lambda i, _n=n: (0,) * _n)


def _make_tc_call(interpret=False):
    gspec = pl.BlockSpec((_BB, EMB_P), lambda i: (i, 0))
    return pl.pallas_call(
        _tc_body,
        grid=(_NB,),
        in_specs=[gspec] * N_FIELDS + [
            pl.BlockSpec((_BB, DENSE), lambda i: (i, 0)),
            _full((32, EMB_P)), _full((32, EMB_P)), _full((32, EMB_P)),
            _full((32, EMB_P)), _full((1, 32)), _full((1, 32)),
            _full((128, EMB_P)), _full((128, EMB_P)), _full((128, EMB_P)),
            _full((128, DENSE)), _full((128, EMB_P)), _full((128, EMB_P)),
            _full((1, 128)), _full((64, 128)), _full((1, 64)),
            _full((1, 64)),
            pl.BlockSpec(memory_space=pltpu.SMEM),
        ],
        out_specs=pl.BlockSpec((1, 1, _BB), lambda i: (i, 0, 0)),
        out_shape=jax.ShapeDtypeStruct((_NB, 1, _BB), jnp.float32),
        interpret=interpret,
    )


def _padw(w):
    return jnp.pad(w, ((0, 0), (0, EMB_P - w.shape[1])))


def _dense_stage(gathered, dense_feature, att_w1, att_b1, att_a, att_w2,
                 att_b2, w1, b1, p1, w2, b2, p2, w3, b3, interpret=False):
    scal = jnp.concatenate([
        att_a.reshape(-1), att_b2.reshape(-1), p1.reshape(-1),
        p2.reshape(-1), b3.reshape(-1),
        jnp.zeros((3,), jnp.float32)]).astype(jnp.float32)
    out = _make_tc_call(interpret)(
        *gathered, dense_feature,
        _padw(att_w1[:, :EMB]), _padw(att_w1[:, EMB:2 * EMB]),
        _padw(att_w1[:, 2 * EMB:3 * EMB]), _padw(att_w1[:, 3 * EMB:]),
        att_b1.reshape(1, 32), att_w2.reshape(1, 32),
        _padw(w1[:, :EMB]), _padw(w1[:, EMB:2 * EMB]),
        _padw(w1[:, 2 * EMB:3 * EMB]),
        w1[:, 3 * EMB:3 * EMB + DENSE],
        _padw(w1[:, 3 * EMB + DENSE:4 * EMB + DENSE]),
        _padw(w1[:, 4 * EMB + DENSE:]),
        b1.reshape(1, 128), w2, b2.reshape(1, 64), w3, scal)
    return out.reshape(-1)


def kernel(sparse_feature, dense_feature, emb_tables, att_w1, att_b1, att_a,
           att_w2, att_b2, w1, b1, p1, w2, b2, p2, w3, b3):
    idx = sparse_feature.astype(jnp.int32).T.reshape(N_FIELDS, _NW, _RPW)
    gathered = []
    for f in range(N_FIELDS):
        tab = _pad_call()(emb_tables[f])
        gathered.append(_sc_gather()(idx[f], tab))
    return _dense_stage(gathered, dense_feature, att_w1, att_b1, att_a,
                        att_w2, att_b2, w1, b1, p1, w2, b2, p2, w3, b3)
